# baseline (speedup 1.0000x reference)
import functools

import jax
import jax.numpy as jnp
from jax import lax
from jax.experimental import pallas as pl
from jax.experimental.pallas import tpu as pltpu
from jax.experimental.pallas import tpu_sc as plsc

NUM_STUDENTS = 1000000
NUM_QUESTIONS = 100000
NUM_DIM = 64
BATCH = 16384

NC = 2
NS = 16
L = 16
NW = NC * NS
B_PER_W = BATCH // NW
CHUNK = 128
N_CHUNKS = B_PER_W // CHUNK

ROWSUM_BW = 65536


def _rowsum_body(xt_ref, o_ref):
    ones = jnp.ones((1, NUM_DIM), jnp.float32)
    o_ref[...] = jnp.dot(ones, xt_ref[...],
                         preferred_element_type=jnp.float32)


def _rowsum(xt):
    n = xt.shape[1]
    grid = (n + ROWSUM_BW - 1) // ROWSUM_BW
    return pl.pallas_call(
        _rowsum_body,
        grid=(grid,),
        in_specs=[pl.BlockSpec((NUM_DIM, ROWSUM_BW), lambda i: (0, i))],
        out_specs=pl.BlockSpec((1, ROWSUM_BW), lambda i: (0, i)),
        out_shape=jax.ShapeDtypeStruct((1, n), jnp.float32),
        compiler_params=pltpu.CompilerParams(
            dimension_semantics=("arbitrary",)),
    )(xt)


def _gather_body(sid_hbm, qid_hbm, ts_hbm, bs_hbm, out_hbm,
                 sid_v, qid_v, ts_v, bs_v, out_v, sem):
    wid = lax.axis_index("s") * NC + lax.axis_index("c")

    pltpu.sync_copy(sid_hbm.at[pl.ds(wid * N_CHUNKS, N_CHUNKS)], sid_v)
    pltpu.sync_copy(qid_hbm.at[pl.ds(wid * N_CHUNKS, N_CHUNKS)], qid_v)

    copies = []
    for j in range(N_CHUNKS):
        copies.append(pltpu.async_copy(ts_hbm.at[sid_v.at[j]], ts_v.at[j], sem))
        copies.append(pltpu.async_copy(bs_hbm.at[qid_v.at[j]], bs_v.at[j], sem))
    for c in copies:
        c.wait()

    for j in range(N_CHUNKS):
        for c in range(CHUNK // L):
            diff = ts_v[j, pl.ds(c * L, L)] - bs_v[j, pl.ds(c * L, L)]
            pred = 1.0 / (1.0 + jnp.exp(-diff))
            out_v[pl.ds((j * (CHUNK // L) + c) * L, L)] = pred

    pltpu.sync_copy(out_v, out_hbm.at[pl.ds(wid * B_PER_W, B_PER_W)])


def _gather_sigmoid(sid2d, qid2d, ts, bs):
    kern = functools.partial(
        pl.kernel,
        mesh=plsc.VectorSubcoreMesh(core_axis_name="c", subcore_axis_name="s"),
        out_type=jax.ShapeDtypeStruct((BATCH,), jnp.float32),
        compiler_params=pltpu.CompilerParams(
            needs_layout_passes=False, use_tc_tiling_on_sc=False),
        scratch_types=[
            pltpu.VMEM((N_CHUNKS, CHUNK), jnp.int32),
            pltpu.VMEM((N_CHUNKS, CHUNK), jnp.int32),
            pltpu.VMEM((N_CHUNKS, CHUNK), jnp.float32),
            pltpu.VMEM((N_CHUNKS, CHUNK), jnp.float32),
            pltpu.VMEM((B_PER_W,), jnp.float32),
            pltpu.SemaphoreType.DMA,
        ],
    )(_gather_body)
    return kern(sid2d, qid2d, ts, bs)


@jax.jit
def _irt(student_ids, question_ids, theta_weight, beta_weight):
    ts = _rowsum(theta_weight.T).reshape(NUM_STUDENTS)
    bs = _rowsum(beta_weight.T).reshape(NUM_QUESTIONS)
    sid2d = student_ids.astype(jnp.int32).reshape(NW * N_CHUNKS, CHUNK)
    qid2d = question_ids.astype(jnp.int32).reshape(NW * N_CHUNKS, CHUNK)
    return _gather_sigmoid(sid2d, qid2d, ts, bs)


def kernel(student_ids, question_ids, theta_weight, beta_weight):
    out = _irt(student_ids, question_ids, theta_weight, beta_weight)
    return out.reshape(BATCH, 1)
